# baseline (speedup 1.0000x reference)
import functools

import jax
import jax.numpy as jnp
from jax import lax
from jax.experimental import pallas as pl
from jax.experimental.pallas import tpu as pltpu
from jax.experimental.pallas import tpu_sc as plsc

TOPK = 8
NUM_EXPERTS = 64
ROW_BLOCK = 1024
N_ROWS = 32768
N_WORKERS = 32
ROWS_PER_W = N_ROWS // N_WORKERS
GROUP = 16


def _mm_block(x_ref, w_ref, b_ref, s_ref):
    s_ref[...] = jnp.dot(x_ref[...], w_ref[...],
                         preferred_element_type=jnp.float32) + b_ref[...]


def _tc_scores(inputs, W, b):
    n_rows = inputs.shape[0]
    return pl.pallas_call(
        _mm_block,
        grid=(n_rows // ROW_BLOCK,),
        in_specs=[
            pl.BlockSpec((ROW_BLOCK, inputs.shape[1]), lambda i: (i, 0)),
            pl.BlockSpec((inputs.shape[1], NUM_EXPERTS), lambda i: (0, 0)),
            pl.BlockSpec((1, NUM_EXPERTS), lambda i: (0, 0)),
        ],
        out_specs=pl.BlockSpec((ROW_BLOCK, NUM_EXPERTS), lambda i: (i, 0)),
        out_shape=jax.ShapeDtypeStruct((n_rows, NUM_EXPERTS), jnp.float32),
    )(inputs, W, b.reshape(1, NUM_EXPERTS))


def _sc_body(scores_hbm, probs_hbm, idx_hbm, sbuf, pbuf, ibuf):
    wid = lax.axis_index("s") * 2 + lax.axis_index("c")
    base = wid * ROWS_PER_W
    row_iota = lax.broadcasted_iota(jnp.int32, (GROUP,), 0)
    gather_base = row_iota * NUM_EXPERTS
    out_base = row_iota * TOPK
    neg_inf = jnp.full((GROUP,), -jnp.inf, jnp.float32)
    zeros_i = jnp.zeros((GROUP,), jnp.int32)

    def group_body(g, _):
        row0 = base + g * GROUP
        pltpu.sync_copy(
            scores_hbm.at[pl.ds(row0 * NUM_EXPERTS, GROUP * NUM_EXPERTS)],
            sbuf)

        def expert_body(e, carry):
            t = list(carry[:TOPK])
            x = list(carry[TOPK:])
            col_idx = jnp.full((GROUP,), e, jnp.int32)
            v = plsc.load_gather(sbuf, [gather_base + e])
            c = [v > t[j] for j in range(TOPK)]
            nt, nx = [], []
            for j in range(TOPK):
                if j == 0:
                    cand_t, cand_x = v, col_idx
                else:
                    cand_t = jnp.where(c[j - 1], t[j - 1], v)
                    cand_x = jnp.where(c[j - 1], x[j - 1], col_idx)
                nt.append(jnp.where(c[j], cand_t, t[j]))
                nx.append(jnp.where(c[j], cand_x, x[j]))
            return tuple(nt) + tuple(nx)

        init = tuple([neg_inf] * TOPK) + tuple([zeros_i] * TOPK)
        carry = lax.fori_loop(0, NUM_EXPERTS, expert_body, init)
        t = carry[:TOPK]
        x = carry[TOPK:]

        es = [jnp.exp(t[j] - t[0]) for j in range(TOPK)]
        s = es[0]
        for j in range(1, TOPK):
            s = s + es[j]
        for j in range(TOPK):
            plsc.store_scatter(pbuf, [out_base + j], es[j] / s)
            plsc.store_scatter(ibuf, [out_base + j], x[j])

        pltpu.sync_copy(pbuf, probs_hbm.at[pl.ds(row0 * TOPK, GROUP * TOPK)])
        pltpu.sync_copy(ibuf, idx_hbm.at[pl.ds(row0 * TOPK, GROUP * TOPK)])
        return 0

    lax.fori_loop(0, ROWS_PER_W // GROUP, group_body, 0)


_sc_router = functools.partial(
    pl.kernel,
    mesh=plsc.VectorSubcoreMesh(core_axis_name="c", subcore_axis_name="s"),
    out_type=[
        jax.ShapeDtypeStruct((N_ROWS * TOPK,), jnp.float32),
        jax.ShapeDtypeStruct((N_ROWS * TOPK,), jnp.int32),
    ],
    scratch_types=[
        pltpu.VMEM((GROUP * NUM_EXPERTS,), jnp.float32),
        pltpu.VMEM((GROUP * TOPK,), jnp.float32),
        pltpu.VMEM((GROUP * TOPK,), jnp.int32),
    ],
    compiler_params=pltpu.CompilerParams(needs_layout_passes=False),
)(_sc_body)


@jax.jit
def kernel(inputs, W, b):
    scores = _tc_scores(inputs, W, b)
    probs_flat, idx_flat = _sc_router(scores.reshape(-1))
    return (probs_flat.reshape(N_ROWS, TOPK), idx_flat.reshape(N_ROWS, TOPK))


# speedup vs baseline: 1.1483x; 1.1483x over previous
import functools

import jax
import jax.numpy as jnp
from jax import lax
from jax.experimental import pallas as pl
from jax.experimental.pallas import tpu as pltpu
from jax.experimental.pallas import tpu_sc as plsc

TOPK = 8
NUM_EXPERTS = 64
ROW_BLOCK = 1024
N_ROWS = 32768
N_WORKERS = 32
ROWS_PER_W = N_ROWS // N_WORKERS
GROUP = 16


def _mm_block(x_ref, w_ref, b_ref, s_ref):
    s_ref[...] = jnp.dot(x_ref[...], w_ref[...],
                         preferred_element_type=jnp.float32) + b_ref[...]


def _tc_scores(inputs, W, b):
    n_rows = inputs.shape[0]
    return pl.pallas_call(
        _mm_block,
        grid=(n_rows // ROW_BLOCK,),
        in_specs=[
            pl.BlockSpec((ROW_BLOCK, inputs.shape[1]), lambda i: (i, 0)),
            pl.BlockSpec((inputs.shape[1], NUM_EXPERTS), lambda i: (0, 0)),
            pl.BlockSpec((1, NUM_EXPERTS), lambda i: (0, 0)),
        ],
        out_specs=pl.BlockSpec((ROW_BLOCK, NUM_EXPERTS), lambda i: (i, 0)),
        out_shape=jax.ShapeDtypeStruct((n_rows, NUM_EXPERTS), jnp.float32),
    )(inputs, W, b.reshape(1, NUM_EXPERTS))


STAGE = 256
PAIR = 2
EUNROLL = 2


def _sc_body(scores_hbm, probs_hbm, idx_hbm, sbuf, pbuf, ibuf):
    wid = lax.axis_index("s") * 2 + lax.axis_index("c")
    base = wid * ROWS_PER_W
    lane = lax.broadcasted_iota(jnp.int32, (GROUP,), 0)
    neg_inf = jnp.full((GROUP,), -jnp.inf, jnp.float32)
    zeros_i = jnp.zeros((GROUP,), jnp.int32)

    def insert(v, e, t, x):
        col_idx = jnp.full((GROUP,), e, jnp.int32)
        c = [v > t[j] for j in range(TOPK)]
        nt, nx = [], []
        for j in range(TOPK):
            if j == 0:
                cand_t, cand_x = v, col_idx
            else:
                cand_t = jnp.where(c[j - 1], t[j - 1], v)
                cand_x = jnp.where(c[j - 1], x[j - 1], col_idx)
            nt.append(jnp.where(c[j], cand_t, t[j]))
            nx.append(jnp.where(c[j], cand_x, x[j]))
        return nt, nx

    def stage_body(s, _):
        row0 = base + s * STAGE
        pltpu.sync_copy(
            scores_hbm.at[pl.ds(row0 * NUM_EXPERTS, STAGE * NUM_EXPERTS)],
            sbuf)

        def sub_body(sub, _):
            rows = [sub * (GROUP * PAIR) + p * GROUP + lane
                    for p in range(PAIR)]
            gbase = [r * NUM_EXPERTS for r in rows]

            def expert_body(i, carry):
                t = [list(carry[p * TOPK:(p + 1) * TOPK]) for p in range(PAIR)]
                x = [list(carry[(PAIR + p) * TOPK:(PAIR + p + 1) * TOPK])
                     for p in range(PAIR)]
                for u in range(EUNROLL):
                    e = i * EUNROLL + u
                    for p in range(PAIR):
                        v = plsc.load_gather(sbuf, [gbase[p] + e])
                        t[p], x[p] = insert(v, e, t[p], x[p])
                return (tuple(t[0]) + tuple(t[1]) + tuple(x[0]) + tuple(x[1]))

            init = (tuple([neg_inf] * (TOPK * PAIR))
                    + tuple([zeros_i] * (TOPK * PAIR)))
            carry = lax.fori_loop(0, NUM_EXPERTS // EUNROLL, expert_body, init)
            for p in range(PAIR):
                t = carry[p * TOPK:(p + 1) * TOPK]
                x = carry[(PAIR + p) * TOPK:(PAIR + p + 1) * TOPK]
                es = [jnp.exp(t[j] - t[0]) for j in range(TOPK)]
                ssum = es[0]
                for j in range(1, TOPK):
                    ssum = ssum + es[j]
                obase = rows[p] * TOPK
                for j in range(TOPK):
                    plsc.store_scatter(pbuf, [obase + j], es[j] / ssum)
                    plsc.store_scatter(ibuf, [obase + j], x[j])
            return 0

        lax.fori_loop(0, STAGE // (GROUP * PAIR), sub_body, 0)
        pltpu.sync_copy(pbuf, probs_hbm.at[pl.ds(row0 * TOPK, STAGE * TOPK)])
        pltpu.sync_copy(ibuf, idx_hbm.at[pl.ds(row0 * TOPK, STAGE * TOPK)])
        return 0

    lax.fori_loop(0, ROWS_PER_W // STAGE, stage_body, 0)


_sc_router = functools.partial(
    pl.kernel,
    mesh=plsc.VectorSubcoreMesh(core_axis_name="c", subcore_axis_name="s"),
    out_type=[
        jax.ShapeDtypeStruct((N_ROWS * TOPK,), jnp.float32),
        jax.ShapeDtypeStruct((N_ROWS * TOPK,), jnp.int32),
    ],
    scratch_types=[
        pltpu.VMEM((STAGE * NUM_EXPERTS,), jnp.float32),
        pltpu.VMEM((STAGE * TOPK,), jnp.float32),
        pltpu.VMEM((STAGE * TOPK,), jnp.int32),
    ],
    compiler_params=pltpu.CompilerParams(needs_layout_passes=False),
)(_sc_body)


@jax.jit
def kernel(inputs, W, b):
    scores = _tc_scores(inputs, W, b)
    probs_flat, idx_flat = _sc_router(scores.reshape(-1))
    return (probs_flat.reshape(N_ROWS, TOPK), idx_flat.reshape(N_ROWS, TOPK))


# speedup vs baseline: 1.2444x; 1.0837x over previous
import functools

import jax
import jax.numpy as jnp
from jax import lax
from jax.experimental import pallas as pl
from jax.experimental.pallas import tpu as pltpu
from jax.experimental.pallas import tpu_sc as plsc

TOPK = 8
NUM_EXPERTS = 64
ROW_BLOCK = 1024
N_ROWS = 32768
N_WORKERS = 32
ROWS_PER_W = N_ROWS // N_WORKERS
GROUP = 16


def _mm_block(x_ref, w_ref, b_ref, s_ref):
    s_ref[...] = jnp.dot(x_ref[...], w_ref[...],
                         preferred_element_type=jnp.float32) + b_ref[...]


def _tc_scores(inputs, W, b):
    n_rows = inputs.shape[0]
    return pl.pallas_call(
        _mm_block,
        grid=(n_rows // ROW_BLOCK,),
        in_specs=[
            pl.BlockSpec((ROW_BLOCK, inputs.shape[1]), lambda i: (i, 0)),
            pl.BlockSpec((inputs.shape[1], NUM_EXPERTS), lambda i: (0, 0)),
            pl.BlockSpec((1, NUM_EXPERTS), lambda i: (0, 0)),
        ],
        out_specs=pl.BlockSpec((ROW_BLOCK, NUM_EXPERTS), lambda i: (i, 0)),
        out_shape=jax.ShapeDtypeStruct((n_rows, NUM_EXPERTS), jnp.float32),
    )(inputs, W, b.reshape(1, NUM_EXPERTS))


STAGE = 256
PAIR = 2
EUNROLL = 2


def _sc_body(scores_hbm, probs_hbm, idx_hbm, sbuf, pbuf, ibuf):
    wid = lax.axis_index("s") * 2 + lax.axis_index("c")
    base = wid * ROWS_PER_W
    lane = lax.broadcasted_iota(jnp.int32, (GROUP,), 0)
    neg_inf = jnp.full((GROUP,), -jnp.inf, jnp.float32)
    zeros_i = jnp.zeros((GROUP,), jnp.int32)

    def insert(v, e, t, x):
        col_idx = jnp.full((GROUP,), e, jnp.int32)
        c = [v > t[j] for j in range(TOPK)]
        nt, nx = [], []
        for j in range(TOPK):
            if j == 0:
                cand_t, cand_x = v, col_idx
            else:
                cand_t = jnp.where(c[j - 1], t[j - 1], v)
                cand_x = jnp.where(c[j - 1], x[j - 1], col_idx)
            nt.append(jnp.where(c[j], cand_t, t[j]))
            nx.append(jnp.where(c[j], cand_x, x[j]))
        return nt, nx

    def stage_body(s, _):
        row0 = base + s * STAGE
        pltpu.sync_copy(
            scores_hbm.at[pl.ds(row0 * NUM_EXPERTS, STAGE * NUM_EXPERTS)],
            sbuf)

        def sub_body(sub, _):
            rows = sub * GROUP + lane
            gbase = rows * NUM_EXPERTS
            t = [neg_inf] * TOPK
            x = [zeros_i] * TOPK
            for e in range(NUM_EXPERTS):
                v = plsc.load_gather(sbuf, [gbase + e])
                t, x = insert(v, e, t, x)
            es = [jnp.exp(t[j] - t[0]) for j in range(TOPK)]
            ssum = es[0]
            for j in range(1, TOPK):
                ssum = ssum + es[j]
            obase = rows * TOPK
            for j in range(TOPK):
                plsc.store_scatter(pbuf, [obase + j], es[j] / ssum)
                plsc.store_scatter(ibuf, [obase + j], x[j])
            return 0

        lax.fori_loop(0, STAGE // GROUP, sub_body, 0)
        pltpu.sync_copy(pbuf, probs_hbm.at[pl.ds(row0 * TOPK, STAGE * TOPK)])
        pltpu.sync_copy(ibuf, idx_hbm.at[pl.ds(row0 * TOPK, STAGE * TOPK)])
        return 0

    lax.fori_loop(0, ROWS_PER_W // STAGE, stage_body, 0)


_sc_router = functools.partial(
    pl.kernel,
    mesh=plsc.VectorSubcoreMesh(core_axis_name="c", subcore_axis_name="s"),
    out_type=[
        jax.ShapeDtypeStruct((N_ROWS * TOPK,), jnp.float32),
        jax.ShapeDtypeStruct((N_ROWS * TOPK,), jnp.int32),
    ],
    scratch_types=[
        pltpu.VMEM((STAGE * NUM_EXPERTS,), jnp.float32),
        pltpu.VMEM((STAGE * TOPK,), jnp.float32),
        pltpu.VMEM((STAGE * TOPK,), jnp.int32),
    ],
    compiler_params=pltpu.CompilerParams(needs_layout_passes=False),
)(_sc_body)


@jax.jit
def kernel(inputs, W, b):
    scores = _tc_scores(inputs, W, b)
    probs_flat, idx_flat = _sc_router(scores.reshape(-1))
    return (probs_flat.reshape(N_ROWS, TOPK), idx_flat.reshape(N_ROWS, TOPK))
